# baseline (speedup 1.0000x reference)
import functools

import jax
import jax.numpy as jnp
from jax import lax
from jax.experimental import pallas as pl
from jax.experimental.pallas import tpu as pltpu
from jax.experimental.pallas import tpu_sc as plsc

B = 16384
D = 64
L = 16
NC = 2
NS = 16
NW = NC * NS
BPW = B // NW
CHUNK = 128
NCHUNK = BPW // CHUNK


def _mf_body(u_hbm, i_hbm, bu_hbm, bi_hbm, p_hbm, q_hbm, out_hbm,
             uidx_v, iidx_v, prow_v, qrow_v, buv_v, biv_v, out_v, sem):
    wid = lax.axis_index("s") * NC + lax.axis_index("c")
    base = wid * BPW

    pltpu.sync_copy(u_hbm.at[wid], uidx_v)
    pltpu.sync_copy(i_hbm.at[wid], iidx_v)

    copies = []
    for j in range(NCHUNK):
        sl = pl.ds(j * CHUNK, CHUNK)
        copies.append(pltpu.async_copy(p_hbm.at[uidx_v.at[j]], prow_v.at[sl], sem))
        copies.append(pltpu.async_copy(q_hbm.at[iidx_v.at[j]], qrow_v.at[sl], sem))
        copies.append(pltpu.async_copy(bu_hbm.at[uidx_v.at[j]], buv_v.at[sl], sem))
        copies.append(pltpu.async_copy(bi_hbm.at[iidx_v.at[j]], biv_v.at[sl], sem))
    for c in copies:
        c.wait()

    lanes = lax.iota(jnp.int32, L)

    def group(g, _):
        vec = jnp.zeros((L,), jnp.float32)
        for rr in range(L):
            r = g * L + rr
            acc = prow_v[r, pl.ds(0, L)] * qrow_v[r, pl.ds(0, L)]
            for k in range(1, D // L):
                acc = acc + prow_v[r, pl.ds(k * L, L)] * qrow_v[r, pl.ds(k * L, L)]
            vec = jnp.where(lanes == rr, jnp.sum(acc), vec)
        sl = pl.ds(g * L, L)
        out_v[sl] = vec + buv_v[sl] + biv_v[sl]
        return _

    lax.fori_loop(0, BPW // L, group, None)

    pltpu.sync_copy(out_v, out_hbm.at[pl.ds(base, BPW)])


_mf = functools.partial(
    pl.kernel,
    out_type=jax.ShapeDtypeStruct((B,), jnp.float32),
    mesh=plsc.VectorSubcoreMesh(core_axis_name="c", subcore_axis_name="s"),
    compiler_params=pltpu.CompilerParams(needs_layout_passes=False, use_tc_tiling_on_sc=False),
    scratch_types=[
        pltpu.VMEM((NCHUNK, CHUNK), jnp.int32),
        pltpu.VMEM((NCHUNK, CHUNK), jnp.int32),
        pltpu.VMEM((BPW, D), jnp.float32),
        pltpu.VMEM((BPW, D), jnp.float32),
        pltpu.VMEM((BPW,), jnp.float32),
        pltpu.VMEM((BPW,), jnp.float32),
        pltpu.VMEM((BPW,), jnp.float32),
        pltpu.SemaphoreType.DMA,
    ],
)(_mf_body)


@jax.jit
def kernel(u_idx, i_idx, mu, b_u, b_i, P, Q):
    u2 = u_idx.astype(jnp.int32).reshape(NW, NCHUNK, CHUNK)
    i2 = i_idx.astype(jnp.int32).reshape(NW, NCHUNK, CHUNK)
    out = _mf(u2, i2, b_u, b_i, P, Q)
    return out + mu
